# baseline (speedup 1.0000x reference)
import functools

import jax
import jax.numpy as jnp
from jax import lax
from jax.experimental import pallas as pl
from jax.experimental.pallas import tpu as pltpu, tpu_sc as plsc

NBUF = 8
SPAD = 56
SENT_PER_CHUNK = 2


def _make_gather(S, R, D, NC, NS):
    NW = NC * NS
    s_per_w = S // NW
    CH = SENT_PER_CHUNK * SPAD
    n_chunks = s_per_w // SENT_PER_CHUNK
    n_groups = n_chunks // NBUF
    mesh = plsc.VectorSubcoreMesh(core_axis_name="c", subcore_axis_name="s")

    @functools.partial(
        pl.kernel,
        mesh=mesh,
        out_type=jax.ShapeDtypeStruct((S, R, D), jnp.float32),
        scratch_types=(
            [
                pltpu.VMEM((s_per_w * SPAD,), jnp.int32),
                pltpu.VMEM((NBUF, CH, D), jnp.float32),
            ]
            + [pltpu.SemaphoreType.DMA] * (2 * NBUF)
        ),
    )
    def k(idx_hbm, table_hbm, out_hbm, idx_v, rows_v, *sems):
        gsem = sems[:NBUF]
        osem = sems[NBUF:]
        wid = lax.axis_index("s") * NC + lax.axis_index("c")
        s_base = wid * s_per_w

        pltpu.sync_copy(idx_hbm.at[pl.ds(s_base * SPAD, s_per_w * SPAD)], idx_v)

        def start_gather(c, b):
            pltpu.async_copy(
                table_hbm.at[idx_v.at[pl.ds(c * CH, CH)]],
                rows_v.at[b],
                gsem[b],
            )

        def wait_gather(b):
            pltpu.make_async_copy(
                table_hbm.at[idx_v.at[pl.ds(0, CH)]],
                rows_v.at[b],
                gsem[b],
            ).wait()

        def start_store(c, b):
            s0 = s_base + c * SENT_PER_CHUNK
            for j in range(SENT_PER_CHUNK):
                pltpu.async_copy(
                    rows_v.at[b].at[pl.ds(j * SPAD, R)],
                    out_hbm.at[s0 + j],
                    osem[b],
                )

        def wait_store(b):
            for j in range(SENT_PER_CHUNK):
                pltpu.make_async_copy(
                    rows_v.at[b].at[pl.ds(j * SPAD, R)],
                    out_hbm.at[0],
                    osem[b],
                ).wait()

        for b in range(NBUF):
            start_gather(b, b)

        def group(g, carry):
            for b in range(NBUF):
                wait_gather(b)
                start_store(g * NBUF + b, b)
            for b in range(NBUF):
                wait_store(b)
                start_gather((g + 1) * NBUF + b, b)
            return carry

        lax.fori_loop(0, n_groups - 1, group, 0)

        last = n_groups - 1
        for b in range(NBUF):
            wait_gather(b)
            start_store(last * NBUF + b, b)
        for b in range(NBUF):
            wait_store(b)

    return k


def kernel(x, weight):
    S, R = x.shape
    D = weight.shape[1]
    idx = jnp.pad(x.astype(jnp.int32), ((0, 0), (0, SPAD - R))).reshape(-1)
    info = plsc.get_sparse_core_info()
    f = _make_gather(S, R, D, info.num_cores, info.num_subcores)
    return f(idx, weight)


# speedup vs baseline: 7.2792x; 7.2792x over previous
import functools

import jax
import jax.numpy as jnp
from jax import lax
from jax.experimental import pallas as pl
from jax.experimental.pallas import tpu as pltpu, tpu_sc as plsc

SPAD = 56
BLK = 8
NBUF = 2


def _make_gather(S, R, D, NC, NS):
    NW = NC * NS
    s_per_w = S // NW
    n_blocks = s_per_w // BLK
    mesh = plsc.VectorSubcoreMesh(core_axis_name="c", subcore_axis_name="s")

    @functools.partial(
        pl.kernel,
        mesh=mesh,
        out_type=jax.ShapeDtypeStruct((S, R, D), jnp.float32),
        scratch_types=(
            [
                pltpu.VMEM((s_per_w * SPAD,), jnp.int32),
                pltpu.VMEM((NBUF, BLK, R, D), jnp.float32),
            ]
            + [pltpu.SemaphoreType.DMA] * (2 * NBUF)
        ),
    )
    def k(idx_hbm, table_hbm, out_hbm, idx_v, rows_v, *sems):
        gsem = sems[:NBUF]
        osem = sems[NBUF:]
        wid = lax.axis_index("s") * NC + lax.axis_index("c")
        s_base = wid * s_per_w

        pltpu.sync_copy(idx_hbm.at[pl.ds(s_base * SPAD, s_per_w * SPAD)], idx_v)

        def start_gathers(blk, b):
            for j in range(BLK):
                pltpu.async_copy(
                    table_hbm.at[idx_v.at[pl.ds((blk * BLK + j) * SPAD, R)]],
                    rows_v.at[b].at[j],
                    gsem[b],
                )

        def wait_gathers(b):
            for j in range(BLK):
                pltpu.make_async_copy(
                    table_hbm.at[idx_v.at[pl.ds(0, R)]],
                    rows_v.at[b].at[j],
                    gsem[b],
                ).wait()

        def start_store(blk, b):
            pltpu.async_copy(
                rows_v.at[b],
                out_hbm.at[pl.ds(s_base + blk * BLK, BLK)],
                osem[b],
            )

        def wait_store(b):
            pltpu.make_async_copy(
                rows_v.at[b],
                out_hbm.at[pl.ds(0, BLK)],
                osem[b],
            ).wait()

        start_gathers(0, 0)
        for blk in range(n_blocks):
            b = blk % NBUF
            nb = (blk + 1) % NBUF
            if blk + 1 < n_blocks:
                if blk >= 1:
                    wait_store(nb)
                start_gathers(blk + 1, nb)
            wait_gathers(b)
            start_store(blk, b)
        wait_store((n_blocks - 1) % NBUF)

    return k


def kernel(x, weight):
    S, R = x.shape
    D = weight.shape[1]
    idx = jnp.pad(x.astype(jnp.int32), ((0, 0), (0, SPAD - R))).reshape(-1)
    info = plsc.get_sparse_core_info()
    f = _make_gather(S, R, D, info.num_cores, info.num_subcores)
    return f(idx, weight)


# speedup vs baseline: 7.3618x; 1.0113x over previous
import functools

import jax
import jax.numpy as jnp
from jax import lax
from jax.experimental import pallas as pl
from jax.experimental.pallas import tpu as pltpu, tpu_sc as plsc

BLK = 8
NBUF = 2
GSPLIT = (104, 104, 104, 88)


def _make_gather(S, R, D, NC, NS):
    NW = NC * NS
    s_per_w = S // NW
    r_per_w = s_per_w * R
    BR = BLK * R
    n_blocks = s_per_w // BLK
    assert sum(GSPLIT) == BR
    mesh = plsc.VectorSubcoreMesh(core_axis_name="c", subcore_axis_name="s")

    @functools.partial(
        pl.kernel,
        mesh=mesh,
        out_type=jax.ShapeDtypeStruct((S, R, D), jnp.float32),
        scratch_types=(
            [
                pltpu.VMEM((r_per_w,), jnp.int32),
                pltpu.VMEM((NBUF, BR, D), jnp.float32),
            ]
            + [pltpu.SemaphoreType.DMA] * (2 * NBUF)
        ),
    )
    def k(idx_hbm, table_hbm, out_hbm, idx_v, rows_v, *sems):
        gsem = sems[:NBUF]
        osem = sems[NBUF:]
        wid = lax.axis_index("s") * NC + lax.axis_index("c")
        s_base = wid * s_per_w

        pltpu.sync_copy(idx_hbm.at[pl.ds(s_base * R, r_per_w)], idx_v)

        def start_gathers(blk, b):
            off = 0
            for n in GSPLIT:
                pltpu.async_copy(
                    table_hbm.at[idx_v.at[pl.ds(blk * BR + off, n)]],
                    rows_v.at[b].at[pl.ds(off, n)],
                    gsem[b],
                )
                off += n

        def wait_gathers(b):
            off = 0
            for n in GSPLIT:
                pltpu.make_async_copy(
                    table_hbm.at[idx_v.at[pl.ds(0, n)]],
                    rows_v.at[b].at[pl.ds(off, n)],
                    gsem[b],
                ).wait()
                off += n

        def start_store(blk, b):
            pltpu.async_copy(
                rows_v.at[b].reshape(BLK, R, D),
                out_hbm.at[pl.ds(s_base + blk * BLK, BLK)],
                osem[b],
            )

        def wait_store(b):
            pltpu.make_async_copy(
                rows_v.at[b].reshape(BLK, R, D),
                out_hbm.at[pl.ds(0, BLK)],
                osem[b],
            ).wait()

        start_gathers(0, 0)
        for blk in range(n_blocks):
            b = blk % NBUF
            nb = (blk + 1) % NBUF
            if blk + 1 < n_blocks:
                if blk >= 1:
                    wait_store(nb)
                start_gathers(blk + 1, nb)
            wait_gathers(b)
            start_store(blk, b)
        wait_store((n_blocks - 1) % NBUF)

    return k


def kernel(x, weight):
    S, R = x.shape
    D = weight.shape[1]
    idx = x.reshape(-1).astype(jnp.int32)
    info = plsc.get_sparse_core_info()
    f = _make_gather(S, R, D, info.num_cores, info.num_subcores)
    return f(idx, weight)


# speedup vs baseline: 7.4942x; 1.0180x over previous
import functools

import jax
import jax.numpy as jnp
from jax import lax
from jax.experimental import pallas as pl
from jax.experimental.pallas import tpu as pltpu, tpu_sc as plsc

BLK = 4
NBUF = 4
GSPLIT = (104, 96)


def _make_gather(S, R, D, NC, NS):
    NW = NC * NS
    s_per_w = S // NW
    r_per_w = s_per_w * R
    BR = BLK * R
    n_blocks = s_per_w // BLK
    assert sum(GSPLIT) == BR
    mesh = plsc.VectorSubcoreMesh(core_axis_name="c", subcore_axis_name="s")

    @functools.partial(
        pl.kernel,
        mesh=mesh,
        out_type=jax.ShapeDtypeStruct((S, R, D), jnp.float32),
        scratch_types=(
            [
                pltpu.VMEM((r_per_w,), jnp.int32),
                pltpu.VMEM((NBUF, BR, D), jnp.float32),
            ]
            + [pltpu.SemaphoreType.DMA] * (2 * NBUF)
        ),
    )
    def k(idx_hbm, table_hbm, out_hbm, idx_v, rows_v, *sems):
        gsem = sems[:NBUF]
        osem = sems[NBUF:]
        wid = lax.axis_index("s") * NC + lax.axis_index("c")
        s_base = wid * s_per_w

        pltpu.sync_copy(idx_hbm.at[pl.ds(s_base * R, r_per_w)], idx_v)

        def start_gathers(blk, b):
            off = 0
            for n in GSPLIT:
                pltpu.async_copy(
                    table_hbm.at[idx_v.at[pl.ds(blk * BR + off, n)]],
                    rows_v.at[b].at[pl.ds(off, n)],
                    gsem[b],
                )
                off += n

        def wait_gathers(b):
            off = 0
            for n in GSPLIT:
                pltpu.make_async_copy(
                    table_hbm.at[idx_v.at[pl.ds(0, n)]],
                    rows_v.at[b].at[pl.ds(off, n)],
                    gsem[b],
                ).wait()
                off += n

        def start_store(blk, b):
            pltpu.async_copy(
                rows_v.at[b].reshape(BLK, R, D),
                out_hbm.at[pl.ds(s_base + blk * BLK, BLK)],
                osem[b],
            )

        def wait_store(b):
            pltpu.make_async_copy(
                rows_v.at[b].reshape(BLK, R, D),
                out_hbm.at[pl.ds(0, BLK)],
                osem[b],
            ).wait()

        start_gathers(0, 0)
        for blk in range(n_blocks):
            b = blk % NBUF
            nb = (blk + 1) % NBUF
            if blk + 1 < n_blocks:
                if blk + 1 >= NBUF:
                    wait_store(nb)
                start_gathers(blk + 1, nb)
            wait_gathers(b)
            start_store(blk, b)
        wait_store((n_blocks - 1) % NBUF)

    return k


def kernel(x, weight):
    S, R = x.shape
    D = weight.shape[1]
    idx = x.reshape(-1).astype(jnp.int32)
    info = plsc.get_sparse_core_info()
    f = _make_gather(S, R, D, info.num_cores, info.num_subcores)
    return f(idx, weight)
